# Initial kernel scaffold; baseline (speedup 1.0000x reference)
import jax
import jax.numpy as jnp
from jax.experimental import pallas as pl


def kernel(x, edge_index, batch, W_in, b_in, W_conv, b_conv, ln_g, ln_b, Wg1, bg1, Wg2, bg2, Wl1, bl1, Wl2, bl2):
    raise NotImplementedError("write your pallas kernel here")


# speedup vs baseline: 8.5074x; 8.5074x over previous
import functools

import jax
import jax.numpy as jnp
from jax import lax
from jax.experimental import pallas as pl
from jax.experimental.pallas import tpu as pltpu
from jax.experimental.pallas import tpu_sc as plsc

N = 10000
E = 320000
D_IN = 256
D_H = 128
L = 3
B = 8

NC = 2
NS = 16
NW = NC * NS
CH = 128
NCHUNK = 79
EPW = NCHUNK * CH
E_PAD = NW * EPW
N_ACC = 10240
ROWS_PER_TILE = N_ACC // NS


def _deg_body(dst_hbm, out_hbm, didx, deg_v):
    c = lax.axis_index("c")
    s = lax.axis_index("s")
    wid = c * NS + s
    zeros16 = jnp.zeros((16,), jnp.float32)
    ones16 = jnp.ones((16,), jnp.float32)

    def _zero(i, carry):
        deg_v[pl.ds(i * 16, 16)] = zeros16
        return carry

    lax.fori_loop(0, N_ACC // 16, _zero, 0)

    def _chunk(j, carry):
        base = pl.multiple_of(wid * EPW + j * CH, CH)
        pltpu.sync_copy(dst_hbm.at[pl.ds(base, CH)], didx)

        def _inner(k, carry2):
            v = didx[pl.ds(k * 16, 16)]
            plsc.addupdate_scatter(deg_v, [v], ones16)
            return carry2

        lax.fori_loop(0, CH // 16, _inner, 0)
        return carry

    lax.fori_loop(0, NCHUNK, _chunk, 0)
    pltpu.sync_copy(deg_v, out_hbm.at[wid])


@functools.cache
def _sc_calls():
    mesh = plsc.VectorSubcoreMesh(core_axis_name="c", subcore_axis_name="s")
    deg = pl.kernel(
        _deg_body,
        out_type=jax.ShapeDtypeStruct((NW, N_ACC), jnp.float32),
        mesh=mesh,
        scratch_types=[
            pltpu.VMEM((CH,), jnp.int32),
            pltpu.VMEM((N_ACC,), jnp.float32),
        ],
        compiler_params=pltpu.CompilerParams(needs_layout_passes=False),
    )
    edge = pl.kernel(
        _edge_body,
        out_type=jax.ShapeDtypeStruct((NC, N_ACC, D_H), jnp.float32),
        mesh=mesh,
        scratch_types=[
            pltpu.VMEM((CH,), jnp.int32),
            pltpu.VMEM((CH,), jnp.int32),
            pltpu.VMEM((CH, D_H), jnp.float32),
            pltpu.VMEM_SHARED((N_ACC, D_H), jnp.float32),
            pltpu.SemaphoreType.DMA,
        ],
    )
    return deg, edge


def _deg_call(dstp):
    return _sc_calls()[0](dstp)


def _edge_call(hws, srcp, dstp, zeros_tile):
    return _sc_calls()[1](hws, srcp, dstp, zeros_tile)


def _edge_body(hws_hbm, src_hbm, dst_hbm, zeros_hbm, out_hbm,
               sidx, didx, rows, acc_sh, sem):
    c = lax.axis_index("c")
    s = lax.axis_index("s")
    wid = c * NS + s

    for i in range(ROWS_PER_TILE // CH):
        pltpu.sync_copy(zeros_hbm, acc_sh.at[pl.ds(s * ROWS_PER_TILE + i * CH, CH)])
    plsc.subcore_barrier()

    def _chunk(j, carry):
        base = pl.multiple_of(wid * EPW + j * CH, CH)
        pltpu.sync_copy(src_hbm.at[pl.ds(base, CH)], sidx)
        pltpu.sync_copy(dst_hbm.at[pl.ds(base, CH)], didx)
        pltpu.async_copy(hws_hbm.at[sidx], rows, sem).wait()
        pltpu.sync_copy(rows, acc_sh.at[didx], add=True)
        return carry

    lax.fori_loop(0, NCHUNK, _chunk, 0)
    plsc.subcore_barrier()
    pltpu.sync_copy(acc_sh.at[pl.ds(s * ROWS_PER_TILE, ROWS_PER_TILE)],
                    out_hbm.at[c, pl.ds(s * ROWS_PER_TILE, ROWS_PER_TILE)])


_BLK = 1000
_NBLK = N // _BLK


def _dot(a, b):
    return jnp.dot(a, b, preferred_element_type=jnp.float32)


def _k1_body(x_ref, win_ref, bin_ref, w0_ref, degt_ref,
             h0_ref, hws_ref, dinv_ref):
    h0 = jnp.maximum(_dot(x_ref[...], win_ref[...]) + bin_ref[...], 0.0)
    deg = 1.0 + jnp.sum(degt_ref[...], axis=1, keepdims=True)
    dinv = lax.rsqrt(jnp.maximum(deg, 1.0))
    h0_ref[...] = h0
    dinv_ref[...] = dinv
    hws_ref[...] = _dot(h0, w0_ref[...]) * dinv


_k1 = pl.pallas_call(
    _k1_body,
    grid=(_NBLK,),
    in_specs=[
        pl.BlockSpec((_BLK, D_IN), lambda i: (i, 0)),
        pl.BlockSpec((D_IN, D_H), lambda i: (0, 0)),
        pl.BlockSpec((1, D_H), lambda i: (0, 0)),
        pl.BlockSpec((D_H, D_H), lambda i: (0, 0)),
        pl.BlockSpec((_BLK, NW), lambda i: (i, 0)),
    ],
    out_specs=[
        pl.BlockSpec((_BLK, D_H), lambda i: (i, 0)),
        pl.BlockSpec((_BLK, D_H), lambda i: (i, 0)),
        pl.BlockSpec((_BLK, 1), lambda i: (i, 0)),
    ],
    out_shape=[
        jax.ShapeDtypeStruct((N, D_H), jnp.float32),
        jax.ShapeDtypeStruct((N, D_H), jnp.float32),
        jax.ShapeDtypeStruct((N, 1), jnp.float32),
    ],
)


def _combine_norm(h_ref, hws_ref, acc_ref, dinv_ref, b_ref, g_ref, beta_ref):
    conv = (acc_ref[0] + acc_ref[1] + hws_ref[...]) * dinv_ref[...] + b_ref[...]
    z = h_ref[...] + conv
    m = jnp.mean(z, axis=1, keepdims=True)
    v = jnp.mean((z - m) ** 2, axis=1, keepdims=True)
    return (z - m) * lax.rsqrt(v + 1e-5) * g_ref[...] + beta_ref[...]


def _k2_body(h_ref, hws_ref, acc_ref, dinv_ref, b_ref, g_ref, beta_ref, wn_ref,
             hout_ref, hwsn_ref):
    hn = _combine_norm(h_ref, hws_ref, acc_ref, dinv_ref, b_ref, g_ref, beta_ref)
    hout_ref[...] = hn
    hwsn_ref[...] = _dot(hn, wn_ref[...]) * dinv_ref[...]


_k2 = pl.pallas_call(
    _k2_body,
    grid=(_NBLK,),
    in_specs=[
        pl.BlockSpec((_BLK, D_H), lambda i: (i, 0)),
        pl.BlockSpec((_BLK, D_H), lambda i: (i, 0)),
        pl.BlockSpec((NC, _BLK, D_H), lambda i: (0, i, 0)),
        pl.BlockSpec((_BLK, 1), lambda i: (i, 0)),
        pl.BlockSpec((1, D_H), lambda i: (0, 0)),
        pl.BlockSpec((1, D_H), lambda i: (0, 0)),
        pl.BlockSpec((1, D_H), lambda i: (0, 0)),
        pl.BlockSpec((D_H, D_H), lambda i: (0, 0)),
    ],
    out_specs=[
        pl.BlockSpec((_BLK, D_H), lambda i: (i, 0)),
        pl.BlockSpec((_BLK, D_H), lambda i: (i, 0)),
    ],
    out_shape=[
        jax.ShapeDtypeStruct((N, D_H), jnp.float32),
        jax.ShapeDtypeStruct((N, D_H), jnp.float32),
    ],
)


def _k3_body(h_ref, hws_ref, acc_ref, dinv_ref, b_ref, g_ref, beta_ref,
             batch_ref, wg1_ref, bg1_ref, wg2_ref, bg2_ref,
             wl1_ref, bl1_ref, wl2_ref, bl2_ref,
             vs_ref, vl_ref, nprob_ref, nlog_ref,
             sum_s, max_s, cnt_s):
    i = pl.program_id(0)
    hn = _combine_norm(h_ref, hws_ref, acc_ref, dinv_ref, b_ref, g_ref, beta_ref)

    nl = _dot(jnp.maximum(_dot(hn, wl1_ref[...]) + bl1_ref[...], 0.0),
              wl2_ref[...]) + bl2_ref[...]
    nlog_ref[...] = nl
    nprob_ref[...] = jax.nn.sigmoid(nl)

    @pl.when(i == 0)
    def _():
        sum_s[...] = jnp.zeros_like(sum_s)
        cnt_s[...] = jnp.zeros_like(cnt_s)
        max_s[...] = jnp.full_like(max_s, -jnp.inf)

    bvec = batch_ref[...]
    for b in range(B):
        mk = bvec == b
        mf = mk.astype(jnp.float32)
        sum_s[pl.ds(b, 1), :] += jnp.sum(hn * mf, axis=0, keepdims=True)
        cnt_s[pl.ds(b, 1), :] += jnp.sum(
            jnp.broadcast_to(mf, (_BLK, D_H)), axis=0, keepdims=True)
        max_s[pl.ds(b, 1), :] = jnp.maximum(
            max_s[pl.ds(b, 1), :],
            jnp.max(jnp.where(mk, hn, -jnp.inf), axis=0, keepdims=True))

    @pl.when(i == pl.num_programs(0) - 1)
    def _():
        mean = sum_s[...] / jnp.maximum(cnt_s[...], 1.0)
        ge = jnp.concatenate([mean, max_s[...]], axis=1)
        vl = _dot(jnp.maximum(_dot(ge, wg1_ref[...]) + bg1_ref[...], 0.0),
                  wg2_ref[...]) + bg2_ref[...]
        vl_ref[...] = vl
        vs_ref[...] = jax.nn.sigmoid(vl)


_k3 = pl.pallas_call(
    _k3_body,
    grid=(_NBLK,),
    in_specs=[
        pl.BlockSpec((_BLK, D_H), lambda i: (i, 0)),
        pl.BlockSpec((_BLK, D_H), lambda i: (i, 0)),
        pl.BlockSpec((NC, _BLK, D_H), lambda i: (0, i, 0)),
        pl.BlockSpec((_BLK, 1), lambda i: (i, 0)),
        pl.BlockSpec((1, D_H), lambda i: (0, 0)),
        pl.BlockSpec((1, D_H), lambda i: (0, 0)),
        pl.BlockSpec((1, D_H), lambda i: (0, 0)),
        pl.BlockSpec((_BLK, 1), lambda i: (i, 0)),
        pl.BlockSpec((2 * D_H, D_H), lambda i: (0, 0)),
        pl.BlockSpec((1, D_H), lambda i: (0, 0)),
        pl.BlockSpec((D_H, 1), lambda i: (0, 0)),
        pl.BlockSpec((1, 1), lambda i: (0, 0)),
        pl.BlockSpec((D_H, D_H // 2), lambda i: (0, 0)),
        pl.BlockSpec((1, D_H // 2), lambda i: (0, 0)),
        pl.BlockSpec((D_H // 2, 1), lambda i: (0, 0)),
        pl.BlockSpec((1, 1), lambda i: (0, 0)),
    ],
    out_specs=[
        pl.BlockSpec((B, 1), lambda i: (0, 0)),
        pl.BlockSpec((B, 1), lambda i: (0, 0)),
        pl.BlockSpec((_BLK, 1), lambda i: (i, 0)),
        pl.BlockSpec((_BLK, 1), lambda i: (i, 0)),
    ],
    out_shape=[
        jax.ShapeDtypeStruct((B, 1), jnp.float32),
        jax.ShapeDtypeStruct((B, 1), jnp.float32),
        jax.ShapeDtypeStruct((N, 1), jnp.float32),
        jax.ShapeDtypeStruct((N, 1), jnp.float32),
    ],
    scratch_shapes=[
        pltpu.VMEM((B, D_H), jnp.float32),
        pltpu.VMEM((B, D_H), jnp.float32),
        pltpu.VMEM((B, D_H), jnp.float32),
    ],
)


@jax.jit
def kernel(x, edge_index, batch, W_in, b_in, W_conv, b_conv, ln_g, ln_b,
           Wg1, bg1, Wg2, bg2, Wl1, bl1, Wl2, bl2):
    pad = E_PAD - E
    srcp = jnp.concatenate([edge_index[0], jnp.zeros((pad,), jnp.int32)])
    dstp = jnp.concatenate([edge_index[1], jnp.full((pad,), N, jnp.int32)])
    zeros_tile = jnp.zeros((CH, D_H), jnp.float32)

    deg_parts = _deg_call(dstp)
    degt = deg_parts.T[:N]

    h, hws, dinv = _k1(x, W_in, b_in.reshape(1, D_H), W_conv[0], degt)

    for l in range(L):
        acc = _edge_call(hws, srcp, dstp, zeros_tile)
        bl = b_conv[l].reshape(1, D_H)
        gl = ln_g[l].reshape(1, D_H)
        bbl = ln_b[l].reshape(1, D_H)
        if l < L - 1:
            h, hws = _k2(h, hws, acc, dinv, bl, gl, bbl, W_conv[l + 1])
        else:
            vs, vlog, nprob, nlog = _k3(
                h, hws, acc, dinv, bl, gl, bbl, batch.reshape(N, 1),
                Wg1, bg1.reshape(1, D_H), Wg2, bg2.reshape(1, 1),
                Wl1, bl1.reshape(1, D_H // 2), Wl2, bl2.reshape(1, 1))

    return (vs[:, 0], vlog[:, 0], nprob[:, 0], nlog[:, 0])
